# baseline (device time: 865497 ns/iter reference)
import jax
import jax.numpy as jnp
from jax import lax
from jax.experimental import pallas as pl
from jax.experimental.pallas import tpu as pltpu

N_DEV = 32
T = 512
D = 256
H = 512
NE = 128
E_LOC = 4
C_OFF = D
A_OFF = D + NE
BLK = D + NE + H


def kernel(x, router_W, route_idx, expert_W):
    def body(x_ref, rw_ref, idx_ref, ew_ref, out_ref,
             comm_ref, w2_ref, send_sems, recv_sems, credit_sem):
        my = lax.axis_index("i")
        left = lax.rem(my - 1 + N_DEV, N_DEV)
        right = lax.rem(my + 1, N_DEV)

        barrier_sem = pltpu.get_barrier_semaphore()
        for nbr in (left, right):
            pl.semaphore_signal(
                barrier_sem, inc=1,
                device_id=(nbr,), device_id_type=pl.DeviceIdType.MESH,
            )
        pl.semaphore_wait(barrier_sem, 2)

        scores = jnp.dot(x_ref[...], rw_ref[...],
                         preferred_element_type=jnp.float32)
        lane = lax.broadcasted_iota(jnp.int32, (T, NE), 1)
        e0 = idx_ref[:, 0:1]
        e1 = idx_ref[:, 1:2]
        oh0 = (lane == e0).astype(jnp.float32)
        oh1 = (lane == e1).astype(jnp.float32)
        s0 = jnp.sum(scores * oh0, axis=1, keepdims=True)
        s1 = jnp.sum(scores * oh1, axis=1, keepdims=True)
        w0 = jax.nn.sigmoid(s0 - s1)
        c = w0 * oh0 + (1.0 - w0) * oh1

        comm_ref[0, :, 0:C_OFF] = x_ref[...]
        comm_ref[0, :, C_OFF:A_OFF] = c
        comm_ref[0, :, A_OFF:BLK] = jnp.zeros((T, H), jnp.float32)

        for k in range(E_LOC):
            w2_ref[:, k * H:(k + 1) * H] = ew_ref[k]

        row = lax.broadcasted_iota(jnp.int32, (NE, E_LOC), 0)
        col = lax.broadcasted_iota(jnp.int32, (NE, E_LOC), 1)
        sel = (row == E_LOC * my + col).astype(jnp.float32)

        for h in range(N_DEV):
            s = h % 2
            r = (h + 1) % 2
            x_b = comm_ref[s, :, 0:C_OFF]
            c_b = comm_ref[s, :, C_OFF:A_OFF]
            coef = jnp.dot(c_b, sel, preferred_element_type=jnp.float32)
            y = jnp.dot(x_b, w2_ref[...], preferred_element_type=jnp.float32)
            contrib = coef[:, 0:1] * y[:, 0:H]
            for k in range(1, E_LOC):
                contrib = contrib + coef[:, k:k + 1] * y[:, k * H:(k + 1) * H]
            comm_ref[s, :, A_OFF:BLK] = comm_ref[s, :, A_OFF:BLK] + contrib

            if h >= 1:
                pl.semaphore_wait(credit_sem, 1)
            rdma = pltpu.make_async_remote_copy(
                src_ref=comm_ref.at[s],
                dst_ref=comm_ref.at[r],
                send_sem=send_sems.at[h],
                recv_sem=recv_sems.at[h],
                device_id=(right,),
                device_id_type=pl.DeviceIdType.MESH,
            )
            rdma.start()
            rdma.wait()
            if h < N_DEV - 1:
                pl.semaphore_signal(
                    credit_sem, inc=1,
                    device_id=(left,), device_id_type=pl.DeviceIdType.MESH,
                )

        out_ref[...] = comm_ref[0, :, A_OFF:BLK]

    return pl.pallas_call(
        body,
        out_shape=jax.ShapeDtypeStruct((T, H), jnp.float32),
        in_specs=[pl.BlockSpec(memory_space=pltpu.VMEM)] * 4,
        out_specs=pl.BlockSpec(memory_space=pltpu.VMEM),
        scratch_shapes=[
            pltpu.VMEM((2, T, BLK), jnp.float32),
            pltpu.VMEM((D, E_LOC * H), jnp.float32),
            pltpu.SemaphoreType.DMA((N_DEV,)),
            pltpu.SemaphoreType.DMA((N_DEV,)),
            pltpu.SemaphoreType.REGULAR,
        ],
        compiler_params=pltpu.CompilerParams(collective_id=0),
    )(x, router_W, route_idx, expert_W)


# device time: 720931 ns/iter; 1.2005x vs baseline; 1.2005x over previous
import jax
import jax.numpy as jnp
from jax import lax
from jax.experimental import pallas as pl
from jax.experimental.pallas import tpu as pltpu

N_DEV = 32
T = 512
TH = T // 2
D = 256
H = 512
NE = 128
E_LOC = 4
C_OFF = D
A_OFF = D + NE
BLK = D + NE + H


def kernel(x, router_W, route_idx, expert_W):
    def body(x_ref, rw_ref, idx_ref, ew_ref, out_ref,
             comm_a, comm_b, w2_ref,
             send_a, recv_a, send_b, recv_b, credit_a, credit_b):
        my = lax.axis_index("i")
        left = lax.rem(my - 1 + N_DEV, N_DEV)
        right = lax.rem(my + 1, N_DEV)

        barrier_sem = pltpu.get_barrier_semaphore()
        for nbr in (left, right):
            pl.semaphore_signal(
                barrier_sem, inc=1,
                device_id=(nbr,), device_id_type=pl.DeviceIdType.MESH,
            )
        pl.semaphore_wait(barrier_sem, 2)

        scores = jnp.dot(x_ref[...], rw_ref[...],
                         preferred_element_type=jnp.float32)
        lane = lax.broadcasted_iota(jnp.int32, (T, NE), 1)
        e0 = idx_ref[:, 0:1]
        e1 = idx_ref[:, 1:2]
        oh0 = (lane == e0).astype(jnp.float32)
        oh1 = (lane == e1).astype(jnp.float32)
        s0 = jnp.sum(scores * oh0, axis=1, keepdims=True)
        s1 = jnp.sum(scores * oh1, axis=1, keepdims=True)
        w0 = jax.nn.sigmoid(s0 - s1)
        c = w0 * oh0 + (1.0 - w0) * oh1

        for comm, lo in ((comm_a, 0), (comm_b, TH)):
            comm[0, :, 0:C_OFF] = x_ref[lo:lo + TH, :]
            comm[0, :, C_OFF:A_OFF] = c[lo:lo + TH, :]
            comm[0, :, A_OFF:BLK] = jnp.zeros((TH, H), jnp.float32)

        for k in range(E_LOC):
            w2_ref[:, k * H:(k + 1) * H] = ew_ref[k]

        row = lax.broadcasted_iota(jnp.int32, (NE, E_LOC), 0)
        col = lax.broadcasted_iota(jnp.int32, (NE, E_LOC), 1)
        sel = (row == E_LOC * my + col).astype(jnp.float32)

        def accumulate(comm, s):
            x_b = comm[s, :, 0:C_OFF]
            c_b = comm[s, :, C_OFF:A_OFF]
            coef = jnp.dot(c_b, sel, preferred_element_type=jnp.float32)
            y = jnp.dot(x_b, w2_ref[...], preferred_element_type=jnp.float32)
            contrib = coef[:, 0:1] * y[:, 0:H]
            for k in range(1, E_LOC):
                contrib = contrib + coef[:, k:k + 1] * y[:, k * H:(k + 1) * H]
            comm[s, :, A_OFF:BLK] = comm[s, :, A_OFF:BLK] + contrib

        for h in range(N_DEV):
            s = h % 2
            r = (h + 1) % 2
            accumulate(comm_a, s)
            accumulate(comm_b, s)
            if h >= 1:
                pl.semaphore_wait(credit_a, 1)
                pl.semaphore_wait(credit_b, 1)
            rdma_a = pltpu.make_async_remote_copy(
                src_ref=comm_a.at[s], dst_ref=comm_a.at[r],
                send_sem=send_a.at[h], recv_sem=recv_a.at[h],
                device_id=(right,), device_id_type=pl.DeviceIdType.MESH,
            )
            rdma_b = pltpu.make_async_remote_copy(
                src_ref=comm_b.at[s], dst_ref=comm_b.at[r],
                send_sem=send_b.at[h], recv_sem=recv_b.at[h],
                device_id=(left,), device_id_type=pl.DeviceIdType.MESH,
            )
            rdma_a.start()
            rdma_b.start()
            rdma_a.wait()
            rdma_b.wait()
            if h < N_DEV - 1:
                pl.semaphore_signal(
                    credit_a, inc=1,
                    device_id=(left,), device_id_type=pl.DeviceIdType.MESH,
                )
                pl.semaphore_signal(
                    credit_b, inc=1,
                    device_id=(right,), device_id_type=pl.DeviceIdType.MESH,
                )

        out_ref[0:TH, :] = comm_a[0, :, A_OFF:BLK]
        out_ref[TH:T, :] = comm_b[0, :, A_OFF:BLK]

    return pl.pallas_call(
        body,
        out_shape=jax.ShapeDtypeStruct((T, H), jnp.float32),
        in_specs=[pl.BlockSpec(memory_space=pltpu.VMEM)] * 4,
        out_specs=pl.BlockSpec(memory_space=pltpu.VMEM),
        scratch_shapes=[
            pltpu.VMEM((2, TH, BLK), jnp.float32),
            pltpu.VMEM((2, TH, BLK), jnp.float32),
            pltpu.VMEM((D, E_LOC * H), jnp.float32),
            pltpu.SemaphoreType.DMA((N_DEV,)),
            pltpu.SemaphoreType.DMA((N_DEV,)),
            pltpu.SemaphoreType.DMA((N_DEV,)),
            pltpu.SemaphoreType.DMA((N_DEV,)),
            pltpu.SemaphoreType.REGULAR,
            pltpu.SemaphoreType.REGULAR,
        ],
        compiler_params=pltpu.CompilerParams(collective_id=0),
    )(x, router_W, route_idx, expert_W)


# device time: 397516 ns/iter; 2.1773x vs baseline; 1.8136x over previous
import jax
import jax.numpy as jnp
from jax import lax
from jax.experimental import pallas as pl
from jax.experimental.pallas import tpu as pltpu

N_DEV = 32
T = 512
TH = T // 2
D = 256
H = 512
NE = 128
E_LOC = 4
C_OFF = D
A_OFF = D + NE
BLK = D + NE + H


def kernel(x, router_W, route_idx, expert_W):
    def body(x_ref, rw_ref, idx_ref, ew_ref, out_ref,
             comm_a, comm_b, w2_ref,
             send_a, recv_a, send_b, recv_b, credit_a, credit_b):
        my = lax.axis_index("i")
        left = lax.rem(my - 1 + N_DEV, N_DEV)
        right = lax.rem(my + 1, N_DEV)

        barrier_sem = pltpu.get_barrier_semaphore()
        for nbr in (left, right):
            pl.semaphore_signal(
                barrier_sem, inc=1,
                device_id=(nbr,), device_id_type=pl.DeviceIdType.MESH,
            )
        pl.semaphore_wait(barrier_sem, 2)

        scores = jnp.dot(x_ref[...], rw_ref[...],
                         preferred_element_type=jnp.float32)
        lane = lax.broadcasted_iota(jnp.int32, (T, NE), 1)
        e0 = idx_ref[:, 0:1]
        e1 = idx_ref[:, 1:2]
        oh0 = (lane == e0).astype(jnp.float32)
        oh1 = (lane == e1).astype(jnp.float32)
        s0 = jnp.sum(scores * oh0, axis=1, keepdims=True)
        s1 = jnp.sum(scores * oh1, axis=1, keepdims=True)
        w0 = jax.nn.sigmoid(s0 - s1)
        c = w0 * oh0 + (1.0 - w0) * oh1

        for comm, lo in ((comm_a, 0), (comm_b, TH)):
            comm[0, :, 0:C_OFF] = x_ref[lo:lo + TH, :].astype(jnp.bfloat16)
            comm[0, :, C_OFF:A_OFF] = c[lo:lo + TH, :].astype(jnp.bfloat16)
            comm[0, :, A_OFF:BLK] = jnp.zeros((TH, H), jnp.bfloat16)

        for k in range(E_LOC):
            w2_ref[:, k * H:(k + 1) * H] = ew_ref[k].astype(jnp.bfloat16)

        row = lax.broadcasted_iota(jnp.int32, (NE, E_LOC), 0)
        col = lax.broadcasted_iota(jnp.int32, (NE, E_LOC), 1)
        sel = (row == E_LOC * my + col).astype(jnp.float32)

        def accumulate(comm, s):
            x_b = comm[s, :, 0:C_OFF]
            c_b = comm[s, :, C_OFF:A_OFF]
            coef = jnp.dot(c_b, sel, preferred_element_type=jnp.float32)
            y = jnp.dot(x_b, w2_ref[...], preferred_element_type=jnp.float32)
            contrib = coef[:, 0:1] * y[:, 0:H]
            for k in range(1, E_LOC):
                contrib = contrib + coef[:, k:k + 1] * y[:, k * H:(k + 1) * H]
            acc = comm[s, :, A_OFF:BLK].astype(jnp.float32) + contrib
            comm[s, :, A_OFF:BLK] = acc.astype(jnp.bfloat16)

        for h in range(N_DEV):
            s = h % 2
            r = (h + 1) % 2
            rdma_a = pltpu.make_async_remote_copy(
                src_ref=comm_a.at[s], dst_ref=comm_a.at[r],
                send_sem=send_a.at[h], recv_sem=recv_a.at[h],
                device_id=(right,), device_id_type=pl.DeviceIdType.MESH,
            )
            rdma_b = pltpu.make_async_remote_copy(
                src_ref=comm_b.at[s], dst_ref=comm_b.at[r],
                send_sem=send_b.at[h], recv_sem=recv_b.at[h],
                device_id=(left,), device_id_type=pl.DeviceIdType.MESH,
            )
            accumulate(comm_a, s)
            if h >= 1:
                pl.semaphore_wait(credit_a, 1)
            rdma_a.start()
            accumulate(comm_b, s)
            if h >= 1:
                pl.semaphore_wait(credit_b, 1)
            rdma_b.start()
            rdma_a.wait()
            rdma_b.wait()
            if h < N_DEV - 1:
                pl.semaphore_signal(
                    credit_a, inc=1,
                    device_id=(left,), device_id_type=pl.DeviceIdType.MESH,
                )
                pl.semaphore_signal(
                    credit_b, inc=1,
                    device_id=(right,), device_id_type=pl.DeviceIdType.MESH,
                )

        out_ref[0:TH, :] = comm_a[0, :, A_OFF:BLK].astype(jnp.float32)
        out_ref[TH:T, :] = comm_b[0, :, A_OFF:BLK].astype(jnp.float32)

    return pl.pallas_call(
        body,
        out_shape=jax.ShapeDtypeStruct((T, H), jnp.float32),
        in_specs=[pl.BlockSpec(memory_space=pltpu.VMEM)] * 4,
        out_specs=pl.BlockSpec(memory_space=pltpu.VMEM),
        scratch_shapes=[
            pltpu.VMEM((2, TH, BLK), jnp.bfloat16),
            pltpu.VMEM((2, TH, BLK), jnp.bfloat16),
            pltpu.VMEM((D, E_LOC * H), jnp.bfloat16),
            pltpu.SemaphoreType.DMA((N_DEV,)),
            pltpu.SemaphoreType.DMA((N_DEV,)),
            pltpu.SemaphoreType.DMA((N_DEV,)),
            pltpu.SemaphoreType.DMA((N_DEV,)),
            pltpu.SemaphoreType.REGULAR,
            pltpu.SemaphoreType.REGULAR,
        ],
        compiler_params=pltpu.CompilerParams(collective_id=0),
    )(x, router_W, route_idx, expert_W)


# device time: 343269 ns/iter; 2.5213x vs baseline; 1.1580x over previous
import jax
import jax.numpy as jnp
from jax import lax
from jax.experimental import pallas as pl
from jax.experimental.pallas import tpu as pltpu

N_DEV = 32
T = 512
TH = T // 2
D = 256
H = 512
NE = 128
E_LOC = 4
C_OFF = D
A_OFF = D + NE
BLK = D + NE + H
LAST = N_DEV - 1


def kernel(x, router_W, route_idx, expert_W):
    def body(x_ref, rw_ref, idx_ref, ew_ref, out_ref,
             comm_a, comm_b, w2_ref,
             xc_send_a, xc_recv_a, ac_send_a, ac_recv_a,
             xc_send_b, xc_recv_b, ac_send_b, ac_recv_b,
             xc_cred_a, ac_cred_a, xc_cred_b, ac_cred_b):
        my = lax.axis_index("i")
        left = lax.rem(my - 1 + N_DEV, N_DEV)
        right = lax.rem(my + 1, N_DEV)

        barrier_sem = pltpu.get_barrier_semaphore()
        for nbr in (left, right):
            pl.semaphore_signal(
                barrier_sem, inc=1,
                device_id=(nbr,), device_id_type=pl.DeviceIdType.MESH,
            )
        pl.semaphore_wait(barrier_sem, 2)

        scores = jnp.dot(x_ref[...], rw_ref[...],
                         preferred_element_type=jnp.float32)
        lane = lax.broadcasted_iota(jnp.int32, (T, NE), 1)
        e0 = idx_ref[:, 0:1]
        e1 = idx_ref[:, 1:2]
        oh0 = (lane == e0).astype(jnp.float32)
        oh1 = (lane == e1).astype(jnp.float32)
        s0 = jnp.sum(scores * oh0, axis=1, keepdims=True)
        s1 = jnp.sum(scores * oh1, axis=1, keepdims=True)
        w0 = jax.nn.sigmoid(s0 - s1)
        c = w0 * oh0 + (1.0 - w0) * oh1

        for comm, lo in ((comm_a, 0), (comm_b, TH)):
            comm[0, :, 0:C_OFF] = x_ref[lo:lo + TH, :].astype(jnp.bfloat16)
            comm[0, :, C_OFF:A_OFF] = c[lo:lo + TH, :].astype(jnp.bfloat16)
            comm[0, :, A_OFF:BLK] = jnp.zeros((TH, H), jnp.bfloat16)

        for k in range(E_LOC):
            w2_ref[:, k * H:(k + 1) * H] = ew_ref[k].astype(jnp.bfloat16)

        row = lax.broadcasted_iota(jnp.int32, (NE, E_LOC), 0)
        col = lax.broadcasted_iota(jnp.int32, (NE, E_LOC), 1)
        sel = (row == E_LOC * my + col).astype(jnp.float32)

        rings = {
            "a": dict(comm=comm_a, xc_send=xc_send_a, xc_recv=xc_recv_a,
                      ac_send=ac_send_a, ac_recv=ac_recv_a,
                      xc_cred=xc_cred_a, ac_cred=ac_cred_a,
                      down=right, up=left, prev=None),
            "b": dict(comm=comm_b, xc_send=xc_send_b, xc_recv=xc_recv_b,
                      ac_send=ac_send_b, ac_recv=ac_recv_b,
                      xc_cred=xc_cred_b, ac_cred=ac_cred_b,
                      down=left, up=right, prev=None),
        }

        def xc_ref(comm, slot):
            return comm.at[slot, :, pl.ds(0, A_OFF)]

        def ac_ref(comm, slot):
            return comm.at[slot, :, pl.ds(A_OFF, H)]

        def cleanup(R, h):
            prev_xc, prev_ac = R["prev"]
            if prev_xc is not None:
                prev_xc.wait_send()
                if h - 1 <= LAST - 2:
                    pl.semaphore_signal(
                        R["xc_cred"], inc=1,
                        device_id=(R["up"],), device_id_type=pl.DeviceIdType.MESH,
                    )
            prev_ac.wait_send()
            if h - 1 <= N_DEV - 2:
                pl.semaphore_signal(
                    R["ac_cred"], inc=1,
                    device_id=(R["up"],), device_id_type=pl.DeviceIdType.MESH,
                )

        def xc_phase(R, h, s, r):
            comm = R["comm"]
            if h >= 1:
                rcv = pltpu.make_async_remote_copy(
                    src_ref=xc_ref(comm, s), dst_ref=xc_ref(comm, s),
                    send_sem=R["xc_send"].at[h - 1],
                    recv_sem=R["xc_recv"].at[h - 1],
                    device_id=(R["up"],), device_id_type=pl.DeviceIdType.MESH,
                )
                rcv.wait_recv()
            if h <= LAST - 1:
                if h >= 1:
                    pl.semaphore_wait(R["xc_cred"], 1)
                fwd = pltpu.make_async_remote_copy(
                    src_ref=xc_ref(comm, s), dst_ref=xc_ref(comm, r),
                    send_sem=R["xc_send"].at[h],
                    recv_sem=R["xc_recv"].at[h],
                    device_id=(R["down"],), device_id_type=pl.DeviceIdType.MESH,
                )
                fwd.start()
            else:
                fwd = None
            return fwd

        def ac_phase(R, h, s, r):
            comm = R["comm"]
            x_b = comm[s, :, 0:C_OFF]
            c_b = comm[s, :, C_OFF:A_OFF]
            coef = jnp.dot(c_b, sel, preferred_element_type=jnp.float32)
            y = jnp.dot(x_b, w2_ref[...], preferred_element_type=jnp.float32)
            contrib = coef[:, 0:1] * y[:, 0:H]
            for k in range(1, E_LOC):
                contrib = contrib + coef[:, k:k + 1] * y[:, k * H:(k + 1) * H]
            if h >= 1:
                rcv = pltpu.make_async_remote_copy(
                    src_ref=ac_ref(comm, s), dst_ref=ac_ref(comm, s),
                    send_sem=R["ac_send"].at[h - 1],
                    recv_sem=R["ac_recv"].at[h - 1],
                    device_id=(R["up"],), device_id_type=pl.DeviceIdType.MESH,
                )
                rcv.wait_recv()
            acc = comm[s, :, A_OFF:BLK].astype(jnp.float32) + contrib
            comm[s, :, A_OFF:BLK] = acc.astype(jnp.bfloat16)
            if h >= 1:
                pl.semaphore_wait(R["ac_cred"], 1)
            snd = pltpu.make_async_remote_copy(
                src_ref=ac_ref(comm, s), dst_ref=ac_ref(comm, r),
                send_sem=R["ac_send"].at[h],
                recv_sem=R["ac_recv"].at[h],
                device_id=(R["down"],), device_id_type=pl.DeviceIdType.MESH,
            )
            snd.start()
            return snd

        for h in range(N_DEV):
            s = h % 2
            r = (h + 1) % 2
            for R in (rings["a"], rings["b"]):
                if h >= 1:
                    cleanup(R, h)
                fwd = xc_phase(R, h, s, r)
                snd = ac_phase(R, h, s, r)
                R["prev"] = (fwd, snd)

        for R in (rings["a"], rings["b"]):
            comm = R["comm"]
            R["prev"][1].wait_send()
            fin = pltpu.make_async_remote_copy(
                src_ref=ac_ref(comm, 0), dst_ref=ac_ref(comm, 0),
                send_sem=R["ac_send"].at[LAST],
                recv_sem=R["ac_recv"].at[LAST],
                device_id=(R["up"],), device_id_type=pl.DeviceIdType.MESH,
            )
            fin.wait_recv()

        out_ref[0:TH, :] = comm_a[0, :, A_OFF:BLK].astype(jnp.float32)
        out_ref[TH:T, :] = comm_b[0, :, A_OFF:BLK].astype(jnp.float32)

    dma = pltpu.SemaphoreType.DMA((N_DEV,))
    return pl.pallas_call(
        body,
        out_shape=jax.ShapeDtypeStruct((T, H), jnp.float32),
        in_specs=[pl.BlockSpec(memory_space=pltpu.VMEM)] * 4,
        out_specs=pl.BlockSpec(memory_space=pltpu.VMEM),
        scratch_shapes=[
            pltpu.VMEM((2, TH, BLK), jnp.bfloat16),
            pltpu.VMEM((2, TH, BLK), jnp.bfloat16),
            pltpu.VMEM((D, E_LOC * H), jnp.bfloat16),
            dma, dma, dma, dma,
            dma, dma, dma, dma,
            pltpu.SemaphoreType.REGULAR,
            pltpu.SemaphoreType.REGULAR,
            pltpu.SemaphoreType.REGULAR,
            pltpu.SemaphoreType.REGULAR,
        ],
        compiler_params=pltpu.CompilerParams(collective_id=0),
    )(x, router_W, route_idx, expert_W)


# device time: 190913 ns/iter; 4.5335x vs baseline; 1.7980x over previous
import jax
import jax.numpy as jnp
from jax import lax
from jax.experimental import pallas as pl
from jax.experimental.pallas import tpu as pltpu

N_DEV = 32
T = 512
TH = T // 2
D = 256
H = 512
NE = 128
E_LOC = 4
C_OFF = D
A_OFF = D + NE
BLK = D + NE + H
LAST = N_DEV - 1


def _build_cycle():
    wmap = {(0, 0): 0, (1, 0): 1, (1, 1): 2, (0, 1): 3,
            (0, 2): 4, (1, 2): 5, (1, 3): 6, (0, 3): 7}
    P = [(0, 0), (1, 0), (2, 0), (3, 0), (3, 1), (2, 1), (1, 1), (0, 1),
         (0, 2), (1, 2), (2, 2), (3, 2), (3, 3), (2, 3), (1, 3), (0, 3)]
    cyc = [(0, y, z) for (y, z) in P] + [(1, y, z) for (y, z) in reversed(P)]
    idx = [8 * z + wmap[(x, y)] for (x, y, z) in cyc]
    assert sorted(idx) == list(range(N_DEV))
    nxt = [0] * N_DEV
    prv = [0] * N_DEV
    for i in range(N_DEV):
        a, b = idx[i], idx[(i + 1) % N_DEV]
        nxt[a] = b
        prv[b] = a
    return nxt, prv


_NEXT, _PREV = _build_cycle()


def kernel(x, router_W, route_idx, expert_W):
    def body(x_ref, rw_ref, idx_ref, ew_ref, nxt_ref, prv_ref, out_ref,
             comm_a, comm_b, w2_ref,
             xc_send_a, xc_recv_a, ac_send_a, ac_recv_a,
             xc_send_b, xc_recv_b, ac_send_b, ac_recv_b,
             xc_cred_a, ac_cred_a, xc_cred_b, ac_cred_b):
        my = lax.axis_index("i")
        nxt = nxt_ref[my]
        prv = prv_ref[my]

        barrier_sem = pltpu.get_barrier_semaphore()
        for nbr in (prv, nxt):
            pl.semaphore_signal(
                barrier_sem, inc=1,
                device_id=(nbr,), device_id_type=pl.DeviceIdType.MESH,
            )
        pl.semaphore_wait(barrier_sem, 2)

        scores = jnp.dot(x_ref[...], rw_ref[...],
                         preferred_element_type=jnp.float32)
        lane = lax.broadcasted_iota(jnp.int32, (T, NE), 1)
        e0 = idx_ref[:, 0:1]
        e1 = idx_ref[:, 1:2]
        oh0 = (lane == e0).astype(jnp.float32)
        oh1 = (lane == e1).astype(jnp.float32)
        s0 = jnp.sum(scores * oh0, axis=1, keepdims=True)
        s1 = jnp.sum(scores * oh1, axis=1, keepdims=True)
        w0 = jax.nn.sigmoid(s0 - s1)
        c = w0 * oh0 + (1.0 - w0) * oh1

        for comm, lo in ((comm_a, 0), (comm_b, TH)):
            comm[0, :, 0:C_OFF] = x_ref[lo:lo + TH, :].astype(jnp.bfloat16)
            comm[0, :, C_OFF:A_OFF] = c[lo:lo + TH, :].astype(jnp.bfloat16)
            comm[0, :, A_OFF:BLK] = jnp.zeros((TH, H), jnp.bfloat16)

        for k in range(E_LOC):
            w2_ref[:, k * H:(k + 1) * H] = ew_ref[k].astype(jnp.bfloat16)

        row = lax.broadcasted_iota(jnp.int32, (NE, E_LOC), 0)
        col = lax.broadcasted_iota(jnp.int32, (NE, E_LOC), 1)
        sel = (row == E_LOC * my + col).astype(jnp.float32)

        rings = {
            "a": dict(comm=comm_a, xc_send=xc_send_a, xc_recv=xc_recv_a,
                      ac_send=ac_send_a, ac_recv=ac_recv_a,
                      xc_cred=xc_cred_a, ac_cred=ac_cred_a,
                      down=nxt, up=prv, prev=None),
            "b": dict(comm=comm_b, xc_send=xc_send_b, xc_recv=xc_recv_b,
                      ac_send=ac_send_b, ac_recv=ac_recv_b,
                      xc_cred=xc_cred_b, ac_cred=ac_cred_b,
                      down=prv, up=nxt, prev=None),
        }

        def xc_ref(comm, slot):
            return comm.at[slot, :, pl.ds(0, A_OFF)]

        def ac_ref(comm, slot):
            return comm.at[slot, :, pl.ds(A_OFF, H)]

        def cleanup(R, h):
            prev_xc, prev_ac = R["prev"]
            if prev_xc is not None:
                prev_xc.wait_send()
                if h - 1 <= LAST - 2:
                    pl.semaphore_signal(
                        R["xc_cred"], inc=1,
                        device_id=(R["up"],), device_id_type=pl.DeviceIdType.MESH,
                    )
            prev_ac.wait_send()
            if h - 1 <= N_DEV - 2:
                pl.semaphore_signal(
                    R["ac_cred"], inc=1,
                    device_id=(R["up"],), device_id_type=pl.DeviceIdType.MESH,
                )

        def xc_phase(R, h, s, r):
            comm = R["comm"]
            if h >= 1:
                rcv = pltpu.make_async_remote_copy(
                    src_ref=xc_ref(comm, s), dst_ref=xc_ref(comm, s),
                    send_sem=R["xc_send"].at[h - 1],
                    recv_sem=R["xc_recv"].at[h - 1],
                    device_id=(R["up"],), device_id_type=pl.DeviceIdType.MESH,
                )
                rcv.wait_recv()
            if h <= LAST - 1:
                if h >= 1:
                    pl.semaphore_wait(R["xc_cred"], 1)
                fwd = pltpu.make_async_remote_copy(
                    src_ref=xc_ref(comm, s), dst_ref=xc_ref(comm, r),
                    send_sem=R["xc_send"].at[h],
                    recv_sem=R["xc_recv"].at[h],
                    device_id=(R["down"],), device_id_type=pl.DeviceIdType.MESH,
                )
                fwd.start()
            else:
                fwd = None
            return fwd

        def ac_phase(R, h, s, r):
            comm = R["comm"]
            x_b = comm[s, :, 0:C_OFF]
            c_b = comm[s, :, C_OFF:A_OFF]
            coef = jnp.dot(c_b, sel, preferred_element_type=jnp.float32)
            y = jnp.dot(x_b, w2_ref[...], preferred_element_type=jnp.float32)
            contrib = coef[:, 0:1] * y[:, 0:H]
            for k in range(1, E_LOC):
                contrib = contrib + coef[:, k:k + 1] * y[:, k * H:(k + 1) * H]
            if h >= 1:
                rcv = pltpu.make_async_remote_copy(
                    src_ref=ac_ref(comm, s), dst_ref=ac_ref(comm, s),
                    send_sem=R["ac_send"].at[h - 1],
                    recv_sem=R["ac_recv"].at[h - 1],
                    device_id=(R["up"],), device_id_type=pl.DeviceIdType.MESH,
                )
                rcv.wait_recv()
            acc = comm[s, :, A_OFF:BLK].astype(jnp.float32) + contrib
            comm[s, :, A_OFF:BLK] = acc.astype(jnp.bfloat16)
            if h >= 1:
                pl.semaphore_wait(R["ac_cred"], 1)
            snd = pltpu.make_async_remote_copy(
                src_ref=ac_ref(comm, s), dst_ref=ac_ref(comm, r),
                send_sem=R["ac_send"].at[h],
                recv_sem=R["ac_recv"].at[h],
                device_id=(R["down"],), device_id_type=pl.DeviceIdType.MESH,
            )
            snd.start()
            return snd

        for h in range(N_DEV):
            s = h % 2
            r = (h + 1) % 2
            for R in (rings["a"], rings["b"]):
                if h >= 1:
                    cleanup(R, h)
                fwd = xc_phase(R, h, s, r)
                snd = ac_phase(R, h, s, r)
                R["prev"] = (fwd, snd)

        for R in (rings["a"], rings["b"]):
            comm = R["comm"]
            R["prev"][1].wait_send()
            fin = pltpu.make_async_remote_copy(
                src_ref=ac_ref(comm, 0), dst_ref=ac_ref(comm, 0),
                send_sem=R["ac_send"].at[LAST],
                recv_sem=R["ac_recv"].at[LAST],
                device_id=(R["up"],), device_id_type=pl.DeviceIdType.MESH,
            )
            fin.wait_recv()

        out_ref[0:TH, :] = comm_a[0, :, A_OFF:BLK].astype(jnp.float32)
        out_ref[TH:T, :] = comm_b[0, :, A_OFF:BLK].astype(jnp.float32)

    dma = pltpu.SemaphoreType.DMA((N_DEV,))
    return pl.pallas_call(
        body,
        out_shape=jax.ShapeDtypeStruct((T, H), jnp.float32),
        in_specs=[pl.BlockSpec(memory_space=pltpu.VMEM)] * 4
        + [pl.BlockSpec(memory_space=pltpu.SMEM)] * 2,
        out_specs=pl.BlockSpec(memory_space=pltpu.VMEM),
        scratch_shapes=[
            pltpu.VMEM((2, TH, BLK), jnp.bfloat16),
            pltpu.VMEM((2, TH, BLK), jnp.bfloat16),
            pltpu.VMEM((D, E_LOC * H), jnp.bfloat16),
            dma, dma, dma, dma,
            dma, dma, dma, dma,
            pltpu.SemaphoreType.REGULAR,
            pltpu.SemaphoreType.REGULAR,
            pltpu.SemaphoreType.REGULAR,
            pltpu.SemaphoreType.REGULAR,
        ],
        compiler_params=pltpu.CompilerParams(collective_id=0),
    )(x, router_W, route_idx, expert_W,
      jnp.asarray(_NEXT, jnp.int32), jnp.asarray(_PREV, jnp.int32))
